# baseline (speedup 1.0000x reference)
import functools

import jax
import jax.numpy as jnp
from jax.experimental import pallas as pl
from jax.experimental.pallas import tpu as pltpu

EPS_GUARD = 0.0001


def _ffn_body(x_ref, m_ref, w1_ref, b1_ref, w2_ref, b2_ref, o_ref):
    p = pl.program_id(1)
    f = pl.program_id(2)
    cb = (f == 0).astype(jnp.float32)

    x = x_ref[...]
    h0 = jnp.dot(x, w1_ref[0], preferred_element_type=jnp.float32)
    h1 = jnp.dot(x, w1_ref[1], preferred_element_type=jnp.float32)
    h0 = jnp.maximum(h0 + b1_ref[0], 0.0).astype(jnp.bfloat16)
    h1 = jnp.maximum(h1 + b1_ref[1], 0.0).astype(jnp.bfloat16)
    part0 = jnp.dot(h0, w2_ref[0], preferred_element_type=jnp.float32)
    part1 = jnp.dot(h1, w2_ref[1], preferred_element_type=jnp.float32)
    contrib = ((part0 + cb * b2_ref[0]) * m_ref[0]
               + (part1 + cb * b2_ref[1]) * m_ref[1])

    @pl.when((p == 0) & (f == 0))
    def _first():
        o_ref[...] = contrib

    @pl.when((p > 0) | (f > 0))
    def _rest():
        o_ref[...] += contrib


def kernel(x, presence, W1, b1, W2, b2):
    N, D = x.shape
    E, _, F = W1.shape

    TN = min(1024, N)
    TF = min(2048, F)
    n_itiles = N // TN
    n_ftiles = F // TF
    n_pairs = E // 2

    mask = (presence.T > EPS_GUARD).astype(jnp.float32)[:, :, None]
    b1r = b1[:, None, :]
    b2r = b2[:, None, :]
    xb = x.astype(jnp.bfloat16)
    W1b = W1.astype(jnp.bfloat16)
    W2b = W2.astype(jnp.bfloat16)

    out = pl.pallas_call(
        _ffn_body,
        grid=(n_itiles, n_pairs, n_ftiles),
        in_specs=[
            pl.BlockSpec((TN, D), lambda i, p, f: (i, 0)),
            pl.BlockSpec((2, TN, 1), lambda i, p, f: (p, i, 0)),
            pl.BlockSpec((2, D, TF), lambda i, p, f: (p, 0, f)),
            pl.BlockSpec((2, 1, TF), lambda i, p, f: (p, 0, f)),
            pl.BlockSpec((2, TF, D), lambda i, p, f: (p, f, 0)),
            pl.BlockSpec((2, 1, D), lambda i, p, f: (p, 0, 0)),
        ],
        out_specs=pl.BlockSpec((TN, D), lambda i, p, f: (i, 0)),
        out_shape=jax.ShapeDtypeStruct((N, D), jnp.float32),
        compiler_params=pltpu.CompilerParams(
            dimension_semantics=("parallel", "arbitrary", "arbitrary"),
        ),
    )(xb, mask, W1b, b1r, W2b, b2r)
    return out
